# baseline (speedup 1.0000x reference)
import jax
import jax.numpy as jnp
from jax.experimental import pallas as pl

_INTERPRET = False

N = 10000
E = 160000
D = 128
NPAD = 10240
NH = 5000
NHPAD = 5120


def _leaky(v):
    return jnp.where(v >= 0, v, 0.1 * v)


def _xs1_body(xp_ref, w_ref, b_ref, o_ref):
    o_ref[...] = jax.nn.relu(
        jnp.dot(xp_ref[...], w_ref[...], preferred_element_type=jnp.float32)
        + b_ref[...])


def _xs1(xp, W_sim, b_sim):
    return pl.pallas_call(
        _xs1_body,
        grid=(5,),
        in_specs=[pl.BlockSpec((2000, 130), lambda i: (i, 0)),
                  pl.BlockSpec((130, 128), lambda i: (0, 0)),
                  pl.BlockSpec((1, 128), lambda i: (0, 0))],
        out_specs=pl.BlockSpec((2000, 128), lambda i: (i, 0)),
        out_shape=jax.ShapeDtypeStruct((N, 128), jnp.float32),
        interpret=_INTERPRET,
    )(xp, W_sim, b_sim.reshape(1, 128))


def _edge_mlp_body(ec_ref, w1_ref, b1_ref, w2_ref, b2_ref, o_ref):
    t = jax.nn.relu(
        jnp.dot(ec_ref[...], w1_ref[...], preferred_element_type=jnp.float32)
        + b1_ref[...])
    o_ref[...] = (jnp.dot(t, w2_ref[...], preferred_element_type=jnp.float32)
                  + b2_ref[...])


def _edge_mlp(ecat, W_e1, b_e1, W_e2, b_e2):
    return pl.pallas_call(
        _edge_mlp_body,
        grid=(40,),
        in_specs=[pl.BlockSpec((4000, 256), lambda i: (i, 0)),
                  pl.BlockSpec((256, 64), lambda i: (0, 0)),
                  pl.BlockSpec((1, 64), lambda i: (0, 0)),
                  pl.BlockSpec((64, 64), lambda i: (0, 0)),
                  pl.BlockSpec((1, 64), lambda i: (0, 0))],
        out_specs=pl.BlockSpec((4000, 64), lambda i: (i, 0)),
        out_shape=jax.ShapeDtypeStruct((E, 64), jnp.float32),
        interpret=_INTERPRET,
    )(ecat, W_e1, b_e1.reshape(1, 64), W_e2, b_e2.reshape(1, 64))


def _kmeans_body(x_ref, c0_ref, cl_ref):
    Xv = x_ref[...]

    def cond(s):
        i, c, a, conv = s
        return (i < 300) & jnp.logical_not(conv)

    def body(s):
        i, c, a, conv = s
        ds = []
        for k in range(4):
            diff = Xv - c[k, :][None, :]
            ds.append(jnp.sum(diff * diff, axis=1, keepdims=True))
        a = jnp.zeros((N, 1), jnp.int32)
        dm = ds[0]
        for k in range(1, 4):
            m = ds[k] < dm
            a = jnp.where(m, k, a)
            dm = jnp.where(m, ds[k], dm)
        rows = []
        cnts = []
        for k in range(4):
            mk = a == k
            rows.append(jnp.sum(jnp.where(mk, Xv, 0.0), axis=0, keepdims=True))
            cnts.append(jnp.sum(jnp.where(mk, 1.0, 0.0), axis=0, keepdims=True))
        sums = jnp.concatenate(rows, axis=0)
        cnt = jnp.concatenate(cnts, axis=0)[:, :1]
        newc = sums / jnp.maximum(cnt, 1.0)
        dc = newc - c
        conv = jnp.sqrt(jnp.sum(dc * dc)) < 1e-4
        return (i + 1, newc, a, conv)

    _, _, a, _ = jax.lax.while_loop(
        cond, body,
        (jnp.int32(0), c0_ref[...], jnp.zeros((N, 1), jnp.int32), jnp.bool_(False)))
    cl_ref[...] = a


def _kmeans(xs2, c0):
    return pl.pallas_call(
        _kmeans_body,
        out_shape=jax.ShapeDtypeStruct((N, 1), jnp.int32),
        interpret=_INTERPRET,
    )(xs2, c0)


def _powers_body(ea_ref, w4_ref, wt_ref, o_ref):
    ea = ea_ref[...]
    wt = wt_ref[...]
    for i in range(4):
        P = jnp.dot(ea, w4_ref[i], preferred_element_type=jnp.float32)
        if i == 0:
            G = P
        else:
            L = _leaky(P)
            G = L
            for _ in range(i - 1):
                G = G * L
        o_ref[i] = G * wt


def _powers(ea, conv4W, w):
    return pl.pallas_call(
        _powers_body,
        grid=(80,),
        in_specs=[pl.BlockSpec((2000, 128), lambda i: (i, 0)),
                  pl.BlockSpec((4, 128, 128), lambda i: (0, 0, 0)),
                  pl.BlockSpec((2000, 1), lambda i: (i, 0))],
        out_specs=pl.BlockSpec((4, 2000, 128), lambda i: (0, i, 0)),
        out_shape=jax.ShapeDtypeStruct((4, E, 128), jnp.float32),
        interpret=_INTERPRET,
    )(ea, conv4W, w.reshape(E, 1))


def _combine_body(s_ref, cl_ref, al_ref, o_ref):
    acc = jnp.zeros(o_ref.shape, jnp.float32)
    cl = cl_ref[...]
    for k in range(4):
        mk = cl == k
        for i in range(4):
            sk = jnp.where(mk, s_ref[i], 0.0)
            acc = acc + jax.lax.dot_general(
                sk, al_ref[k, i], (((1,), (1,)), ((), ())),
                preferred_element_type=jnp.float32)
    o_ref[...] = acc


def _combine(S, cluster_pad, alpha4):
    return pl.pallas_call(
        _combine_body,
        grid=(8,),
        in_specs=[pl.BlockSpec((4, 1280, 128), lambda i: (0, i, 0)),
                  pl.BlockSpec((1280, 1), lambda i: (i, 0)),
                  pl.BlockSpec((4, 4, 128, 128), lambda i: (0, 0, 0, 0))],
        out_specs=pl.BlockSpec((1280, 128), lambda i: (i, 0)),
        out_shape=jax.ShapeDtypeStruct((NPAD, 128), jnp.float32),
        interpret=_INTERPRET,
    )(S, cluster_pad, alpha4)


def _top4_body(qx_ref, qy_ref, px_ref, py_ref, o0, o1, o2, o3):
    qx = qx_ref[...]
    qy = qy_ref[...]
    dx = qx - px_ref[...]
    dy = qy - py_ref[...]
    d = jnp.sqrt(dx * dx + dy * dy)
    iota = jax.lax.broadcasted_iota(jnp.int32, d.shape, 1)
    outs = [o0, o1, o2, o3]
    for j in range(4):
        m = jnp.min(d, axis=1, keepdims=True)
        idx = jnp.min(jnp.where(d == m, iota, 2147483647), axis=1, keepdims=True)
        outs[j][...] = idx
        if j < 3:
            d = jnp.where(iota == idx, jnp.inf, d)


def _top4(phx, phy, pxp, pyp):
    shp = jax.ShapeDtypeStruct((NHPAD, 1), jnp.int32)
    return pl.pallas_call(
        _top4_body,
        grid=(40,),
        in_specs=[pl.BlockSpec((128, 1), lambda i: (i, 0)),
                  pl.BlockSpec((128, 1), lambda i: (i, 0)),
                  pl.BlockSpec((1, NPAD), lambda i: (0, 0)),
                  pl.BlockSpec((1, NPAD), lambda i: (0, 0))],
        out_specs=[pl.BlockSpec((128, 1), lambda i: (i, 0))] * 4,
        out_shape=[shp, shp, shp, shp],
        interpret=_INTERPRET,
    )(phx, phy, pxp, pyp)


def _final_body(eh_ref, xh_ref, wa_ref, wb_ref, b_ref, o_ref):
    o_ref[...] = (
        jnp.dot(eh_ref[...], wa_ref[...], preferred_element_type=jnp.float32)
        + jnp.dot(xh_ref[...], wb_ref[...], preferred_element_type=jnp.float32)
        + b_ref[...])


def _final(ehp, xhp, W3, b3):
    return pl.pallas_call(
        _final_body,
        grid=(4,),
        in_specs=[pl.BlockSpec((1280, 128), lambda i: (i, 0)),
                  pl.BlockSpec((1280, 128), lambda i: (i, 0)),
                  pl.BlockSpec((128, 128), lambda i: (0, 0)),
                  pl.BlockSpec((128, 128), lambda i: (0, 0)),
                  pl.BlockSpec((1, 128), lambda i: (0, 0))],
        out_specs=pl.BlockSpec((1280, 128), lambda i: (i, 0)),
        out_shape=jax.ShapeDtypeStruct((NHPAD, 128), jnp.float32),
        interpret=_INTERPRET,
    )(ehp, xhp, W3[:128], W3[128:], b3.reshape(1, 128))


def kernel(x, edge_index, edge_attr, pos, edge_index_high, edge_attr_high, pos_high, W_sim, b_sim, W_e1, b_e1, W_e2, b_e2, conv1W, alpha1, conv4W, alpha4, W3, b3):
    n = x.shape[0]
    src, dst = edge_index[0], edge_index[1]

    xp = jnp.concatenate([x, pos], axis=1)
    xs1 = _xs1(xp, W_sim, b_sim)

    xi = xs1[dst]
    xj = xs1[src]
    ecat = jnp.concatenate([xi, xj - xi], axis=1)

    h = _edge_mlp(ecat, W_e1, b_e1, W_e2, b_e2)

    xs2 = jnp.zeros((n, 64), jnp.float32).at[dst].max(h)

    init = jax.random.randint(jax.random.key(42), (4,), 0, n)
    c0 = xs2[init]
    cluster = _kmeans(xs2, c0)[:, 0]

    deg = jax.ops.segment_sum(jnp.ones((src.shape[0],), jnp.float32), src, num_segments=n)
    rdeg = 1.0 / jnp.maximum(deg, 1.0)
    w = jnp.where(cluster[src] == cluster[dst], rdeg[src], 0.0)

    SG = _powers(edge_attr, conv4W, w)

    S = jnp.zeros((4, NPAD, 128), jnp.float32).at[:, dst].add(SG)

    cluster_pad = jnp.pad(cluster, (0, NPAD - n), constant_values=-1).reshape(NPAD, 1)
    e = _combine(S, cluster_pad, alpha4)

    phx = jnp.pad(pos_high[:, 0], (0, NHPAD - NH)).reshape(NHPAD, 1)
    phy = jnp.pad(pos_high[:, 1], (0, NHPAD - NH)).reshape(NHPAD, 1)
    pxp = jnp.pad(pos[:, 0], (0, NPAD - n), constant_values=1e30).reshape(1, NPAD)
    pyp = jnp.pad(pos[:, 1], (0, NPAD - n), constant_values=1e30).reshape(1, NPAD)
    i0, i1, i2, i3 = _top4(phx, phy, pxp, pyp)
    idx = jnp.concatenate([i0, i1, i2, i3], axis=1)

    ehp = jnp.mean(e[idx], axis=1)
    xg = jnp.mean(x[idx[:NH]], axis=1)
    xhp = jnp.pad(xg, ((0, NHPAD - NH), (0, 0)))

    out = _final(ehp, xhp, W3, b3)
    return out[:NH]


# speedup vs baseline: 3.9711x; 3.9711x over previous
import functools

import jax
import jax.numpy as jnp
from jax import lax
from jax.experimental import pallas as pl
from jax.experimental.pallas import tpu as pltpu
from jax.experimental.pallas import tpu_sc as plsc

_INTERPRET = False

N = 10000
E = 160000
D = 128
NPAD = 10240
NH = 5000
NHPAD = 5120


def _leaky(v):
    return jnp.where(v >= 0, v, 0.1 * v)


def _xs1_body(xp_ref, w_ref, b_ref, o_ref):
    o_ref[...] = jax.nn.relu(
        jnp.dot(xp_ref[...], w_ref[...], preferred_element_type=jnp.float32)
        + b_ref[...])


def _xs1(xp, W_sim, b_sim):
    return pl.pallas_call(
        _xs1_body,
        grid=(5,),
        in_specs=[pl.BlockSpec((2000, 130), lambda i: (i, 0)),
                  pl.BlockSpec((130, 128), lambda i: (0, 0)),
                  pl.BlockSpec((1, 128), lambda i: (0, 0))],
        out_specs=pl.BlockSpec((2000, 128), lambda i: (i, 0)),
        out_shape=jax.ShapeDtypeStruct((N, 128), jnp.float32),
        interpret=_INTERPRET,
    )(xp, W_sim, b_sim.reshape(1, 128))


def _edge_mlp_body(ec_ref, w1_ref, b1_ref, w2_ref, b2_ref, o_ref):
    t = jax.nn.relu(
        jnp.dot(ec_ref[...], w1_ref[...], preferred_element_type=jnp.float32)
        + b1_ref[...])
    o_ref[...] = (jnp.dot(t, w2_ref[...], preferred_element_type=jnp.float32)
                  + b2_ref[...])


def _edge_mlp(ecat, W_e1, b_e1, W_e2, b_e2):
    return pl.pallas_call(
        _edge_mlp_body,
        grid=(40,),
        in_specs=[pl.BlockSpec((4000, 256), lambda i: (i, 0)),
                  pl.BlockSpec((256, 64), lambda i: (0, 0)),
                  pl.BlockSpec((1, 64), lambda i: (0, 0)),
                  pl.BlockSpec((64, 64), lambda i: (0, 0)),
                  pl.BlockSpec((1, 64), lambda i: (0, 0))],
        out_specs=pl.BlockSpec((4000, 64), lambda i: (i, 0)),
        out_shape=jax.ShapeDtypeStruct((E, 64), jnp.float32),
        interpret=_INTERPRET,
    )(ecat, W_e1, b_e1.reshape(1, 64), W_e2, b_e2.reshape(1, 64))


def _kmeans_body(x_ref, c0_ref, cl_ref):
    Xv = x_ref[...]

    def cond(s):
        i, c, a, conv = s
        return (i < 300) & jnp.logical_not(conv)

    def body(s):
        i, c, a, conv = s
        ds = []
        for k in range(4):
            diff = Xv - c[k, :][None, :]
            ds.append(jnp.sum(diff * diff, axis=1, keepdims=True))
        a = jnp.zeros((N, 1), jnp.int32)
        dm = ds[0]
        for k in range(1, 4):
            m = ds[k] < dm
            a = jnp.where(m, k, a)
            dm = jnp.where(m, ds[k], dm)
        rows = []
        cnts = []
        for k in range(4):
            mk = a == k
            rows.append(jnp.sum(jnp.where(mk, Xv, 0.0), axis=0, keepdims=True))
            cnts.append(jnp.sum(jnp.where(mk, 1.0, 0.0), axis=0, keepdims=True))
        sums = jnp.concatenate(rows, axis=0)
        cnt = jnp.concatenate(cnts, axis=0)[:, :1]
        newc = sums / jnp.maximum(cnt, 1.0)
        dc = newc - c
        conv = jnp.sqrt(jnp.sum(dc * dc)) < 1e-4
        return (i + 1, newc, a, conv)

    _, _, a, _ = jax.lax.while_loop(
        cond, body,
        (jnp.int32(0), c0_ref[...], jnp.zeros((N, 1), jnp.int32), jnp.bool_(False)))
    cl_ref[...] = a


def _kmeans(xs2, c0):
    return pl.pallas_call(
        _kmeans_body,
        out_shape=jax.ShapeDtypeStruct((N, 1), jnp.int32),
        interpret=_INTERPRET,
    )(xs2, c0)


def _powers_body(ea_ref, w4_ref, wt_ref, o_ref):
    ea = ea_ref[...]
    wt = wt_ref[...]
    for i in range(4):
        P = jnp.dot(ea, w4_ref[i], preferred_element_type=jnp.float32)
        if i == 0:
            G = P
        else:
            L = _leaky(P)
            G = L
            for _ in range(i - 1):
                G = G * L
        o_ref[i] = G * wt


def _powers(ea, conv4W, w):
    return pl.pallas_call(
        _powers_body,
        grid=(80,),
        in_specs=[pl.BlockSpec((2000, 128), lambda i: (i, 0)),
                  pl.BlockSpec((4, 128, 128), lambda i: (0, 0, 0)),
                  pl.BlockSpec((2000, 1), lambda i: (i, 0))],
        out_specs=pl.BlockSpec((4, 2000, 128), lambda i: (0, i, 0)),
        out_shape=jax.ShapeDtypeStruct((4, E, 128), jnp.float32),
        interpret=_INTERPRET,
    )(ea, conv4W, w.reshape(E, 1))


def _scatter_add_sc(SG, dst, zeros):
    mesh = plsc.VectorSubcoreMesh(core_axis_name="c", subcore_axis_name="s")

    @functools.partial(
        pl.kernel, mesh=mesh,
        out_type=jax.ShapeDtypeStruct((2, 4, NPAD, 128), jnp.float32),
        scratch_types=[
            pltpu.VMEM((128,), jnp.int32),
            pltpu.VMEM((128, 128), jnp.float32),
            pltpu.VMEM_SHARED((NPAD, 128), jnp.float32),
            pltpu.SemaphoreType.DMA,
        ],
    )
    def k(sg_hbm, dst_hbm, z_hbm, out_hbm, idx_v, rows_v, acc_sh, sem):
        c = lax.axis_index("c")
        s = lax.axis_index("s")
        wid = s * 2 + c
        nb = 39 + jnp.where(wid < 2, 1, 0)
        for i in range(4):
            pltpu.sync_copy(z_hbm, acc_sh.at[pl.ds(s * 640, 640)])
            plsc.subcore_barrier()

            def body(j, carry):
                bi = wid + 32 * j
                pltpu.sync_copy(dst_hbm.at[pl.ds(bi * 128, 128)], idx_v)
                pltpu.sync_copy(sg_hbm.at[i, pl.ds(bi * 128, 128)], rows_v)
                pltpu.sync_copy(rows_v, acc_sh.at[idx_v], add=True)
                return carry

            lax.fori_loop(0, nb, body, 0)
            plsc.subcore_barrier()
            pltpu.sync_copy(acc_sh.at[pl.ds(s * 640, 640)],
                            out_hbm.at[c, i, pl.ds(s * 640, 640)])
            plsc.subcore_barrier()

    return k(SG, dst, zeros)


def _combine_body(s_ref, cl_ref, al_ref, o_ref):
    acc = jnp.zeros(o_ref.shape, jnp.float32)
    cl = cl_ref[...]
    for k in range(4):
        mk = cl == k
        for i in range(4):
            sk = jnp.where(mk, s_ref[0, i] + s_ref[1, i], 0.0)
            acc = acc + jax.lax.dot_general(
                sk, al_ref[k, i], (((1,), (1,)), ((), ())),
                preferred_element_type=jnp.float32)
    o_ref[...] = acc


def _combine(S, cluster_pad, alpha4):
    return pl.pallas_call(
        _combine_body,
        grid=(8,),
        in_specs=[pl.BlockSpec((2, 4, 1280, 128), lambda i: (0, 0, i, 0)),
                  pl.BlockSpec((1280, 1), lambda i: (i, 0)),
                  pl.BlockSpec((4, 4, 128, 128), lambda i: (0, 0, 0, 0))],
        out_specs=pl.BlockSpec((1280, 128), lambda i: (i, 0)),
        out_shape=jax.ShapeDtypeStruct((NPAD, 128), jnp.float32),
        interpret=_INTERPRET,
    )(S, cluster_pad, alpha4)


def _top4_body(qx_ref, qy_ref, px_ref, py_ref, o0, o1, o2, o3):
    qx = qx_ref[...]
    qy = qy_ref[...]
    dx = qx - px_ref[...]
    dy = qy - py_ref[...]
    d = jnp.sqrt(dx * dx + dy * dy)
    iota = jax.lax.broadcasted_iota(jnp.int32, d.shape, 1)
    outs = [o0, o1, o2, o3]
    for j in range(4):
        m = jnp.min(d, axis=1, keepdims=True)
        idx = jnp.min(jnp.where(d == m, iota, 2147483647), axis=1, keepdims=True)
        outs[j][...] = idx
        if j < 3:
            d = jnp.where(iota == idx, jnp.inf, d)


def _top4(phx, phy, pxp, pyp):
    shp = jax.ShapeDtypeStruct((NHPAD, 1), jnp.int32)
    return pl.pallas_call(
        _top4_body,
        grid=(40,),
        in_specs=[pl.BlockSpec((128, 1), lambda i: (i, 0)),
                  pl.BlockSpec((128, 1), lambda i: (i, 0)),
                  pl.BlockSpec((1, NPAD), lambda i: (0, 0)),
                  pl.BlockSpec((1, NPAD), lambda i: (0, 0))],
        out_specs=[pl.BlockSpec((128, 1), lambda i: (i, 0))] * 4,
        out_shape=[shp, shp, shp, shp],
        interpret=_INTERPRET,
    )(phx, phy, pxp, pyp)


def _final_body(eh_ref, xh_ref, wa_ref, wb_ref, b_ref, o_ref):
    o_ref[...] = (
        jnp.dot(eh_ref[...], wa_ref[...], preferred_element_type=jnp.float32)
        + jnp.dot(xh_ref[...], wb_ref[...], preferred_element_type=jnp.float32)
        + b_ref[...])


def _final(ehp, xhp, W3, b3):
    return pl.pallas_call(
        _final_body,
        grid=(4,),
        in_specs=[pl.BlockSpec((1280, 128), lambda i: (i, 0)),
                  pl.BlockSpec((1280, 128), lambda i: (i, 0)),
                  pl.BlockSpec((128, 128), lambda i: (0, 0)),
                  pl.BlockSpec((128, 128), lambda i: (0, 0)),
                  pl.BlockSpec((1, 128), lambda i: (0, 0))],
        out_specs=pl.BlockSpec((1280, 128), lambda i: (i, 0)),
        out_shape=jax.ShapeDtypeStruct((NHPAD, 128), jnp.float32),
        interpret=_INTERPRET,
    )(ehp, xhp, W3[:128], W3[128:], b3.reshape(1, 128))


def kernel(x, edge_index, edge_attr, pos, edge_index_high, edge_attr_high, pos_high, W_sim, b_sim, W_e1, b_e1, W_e2, b_e2, conv1W, alpha1, conv4W, alpha4, W3, b3):
    n = x.shape[0]
    src, dst = edge_index[0], edge_index[1]

    xp = jnp.concatenate([x, pos], axis=1)
    xs1 = _xs1(xp, W_sim, b_sim)

    xi = xs1[dst]
    xj = xs1[src]
    ecat = jnp.concatenate([xi, xj - xi], axis=1)

    h = _edge_mlp(ecat, W_e1, b_e1, W_e2, b_e2)

    xs2 = jnp.zeros((n, 64), jnp.float32).at[dst].max(h)

    init = jax.random.randint(jax.random.key(42), (4,), 0, n)
    c0 = xs2[init]
    cluster = _kmeans(xs2, c0)[:, 0]

    deg = jax.ops.segment_sum(jnp.ones((src.shape[0],), jnp.float32), src, num_segments=n)
    rdeg = 1.0 / jnp.maximum(deg, 1.0)
    w = jnp.where(cluster[src] == cluster[dst], rdeg[src], 0.0)

    SG = _powers(edge_attr, conv4W, w)

    S = _scatter_add_sc(SG, dst, jnp.zeros((640, 128), jnp.float32))

    cluster_pad = jnp.pad(cluster, (0, NPAD - n), constant_values=-1).reshape(NPAD, 1)
    e = _combine(S, cluster_pad, alpha4)

    phx = jnp.pad(pos_high[:, 0], (0, NHPAD - NH)).reshape(NHPAD, 1)
    phy = jnp.pad(pos_high[:, 1], (0, NHPAD - NH)).reshape(NHPAD, 1)
    pxp = jnp.pad(pos[:, 0], (0, NPAD - n), constant_values=1e30).reshape(1, NPAD)
    pyp = jnp.pad(pos[:, 1], (0, NPAD - n), constant_values=1e30).reshape(1, NPAD)
    i0, i1, i2, i3 = _top4(phx, phy, pxp, pyp)
    idx = jnp.concatenate([i0, i1, i2, i3], axis=1)

    ehp = jnp.mean(e[idx], axis=1)
    xg = jnp.mean(x[idx[:NH]], axis=1)
    xhp = jnp.pad(xg, ((0, NHPAD - NH), (0, 0)))

    out = _final(ehp, xhp, W3, b3)
    return out[:NH]


# speedup vs baseline: 9.7650x; 2.4590x over previous
import functools

import jax
import jax.numpy as jnp
from jax import lax
from jax.experimental import pallas as pl
from jax.experimental.pallas import tpu as pltpu
from jax.experimental.pallas import tpu_sc as plsc

_INTERPRET = False

N = 10000
E = 160000
D = 128
NPAD = 10240
NH = 5000
NHPAD = 5120


def _leaky(v):
    return jnp.where(v >= 0, v, 0.1 * v)


def _xs1_body(xp_ref, w_ref, b_ref, o_ref):
    o_ref[...] = jax.nn.relu(
        jnp.dot(xp_ref[...], w_ref[...], preferred_element_type=jnp.float32)
        + b_ref[...])


def _xs1(xp, W_sim, b_sim):
    return pl.pallas_call(
        _xs1_body,
        grid=(5,),
        in_specs=[pl.BlockSpec((2000, 130), lambda i: (i, 0)),
                  pl.BlockSpec((130, 128), lambda i: (0, 0)),
                  pl.BlockSpec((1, 128), lambda i: (0, 0))],
        out_specs=pl.BlockSpec((2000, 128), lambda i: (i, 0)),
        out_shape=jax.ShapeDtypeStruct((N, 128), jnp.float32),
        interpret=_INTERPRET,
    )(xp, W_sim, b_sim.reshape(1, 128))


def _edge_mlp_body(xi_ref, xj_ref, w1_ref, b1_ref, w2_ref, b2_ref, o_ref):
    xi = xi_ref[...]
    ec = jnp.concatenate([xi, xj_ref[...] - xi], axis=1)
    t = jax.nn.relu(
        jnp.dot(ec, w1_ref[...], preferred_element_type=jnp.float32)
        + b1_ref[...])
    o_ref[...] = (jnp.dot(t, w2_ref[...], preferred_element_type=jnp.float32)
                  + b2_ref[...])


def _edge_mlp(XI, XJ, W_e1, b_e1, W_e2, b_e2):
    return pl.pallas_call(
        _edge_mlp_body,
        grid=(40,),
        in_specs=[pl.BlockSpec((4000, 128), lambda i: (i, 0)),
                  pl.BlockSpec((4000, 128), lambda i: (i, 0)),
                  pl.BlockSpec((256, 64), lambda i: (0, 0)),
                  pl.BlockSpec((1, 64), lambda i: (0, 0)),
                  pl.BlockSpec((64, 64), lambda i: (0, 0)),
                  pl.BlockSpec((1, 64), lambda i: (0, 0))],
        out_specs=pl.BlockSpec((4000, 64), lambda i: (i, 0)),
        out_shape=jax.ShapeDtypeStruct((E, 64), jnp.float32),
        interpret=_INTERPRET,
    )(XI, XJ, W_e1, b_e1.reshape(1, 64), W_e2, b_e2.reshape(1, 64))


def _kmeans_body(x_ref, c0_ref, cl_ref):
    Xv = x_ref[...]

    def cond(s):
        i, c, a, conv = s
        return (i < 300) & jnp.logical_not(conv)

    def body(s):
        i, c, a, conv = s
        ds = []
        for k in range(4):
            diff = Xv - c[k, :][None, :]
            ds.append(jnp.sum(diff * diff, axis=1, keepdims=True))
        a = jnp.zeros((N, 1), jnp.int32)
        dm = ds[0]
        for k in range(1, 4):
            m = ds[k] < dm
            a = jnp.where(m, k, a)
            dm = jnp.where(m, ds[k], dm)
        rows = []
        cnts = []
        for k in range(4):
            mk = a == k
            rows.append(jnp.sum(jnp.where(mk, Xv, 0.0), axis=0, keepdims=True))
            cnts.append(jnp.sum(jnp.where(mk, 1.0, 0.0), axis=0, keepdims=True))
        sums = jnp.concatenate(rows, axis=0)
        cnt = jnp.concatenate(cnts, axis=0)[:, :1]
        newc = sums / jnp.maximum(cnt, 1.0)
        dc = newc - c
        conv = jnp.sqrt(jnp.sum(dc * dc)) < 1e-4
        return (i + 1, newc, a, conv)

    _, _, a, _ = jax.lax.while_loop(
        cond, body,
        (jnp.int32(0), c0_ref[...], jnp.zeros((N, 1), jnp.int32), jnp.bool_(False)))
    cl_ref[...] = a


def _kmeans(xs2, c0):
    return pl.pallas_call(
        _kmeans_body,
        out_shape=jax.ShapeDtypeStruct((N, 1), jnp.int32),
        interpret=_INTERPRET,
    )(xs2, c0)


def _powers_body(ea_ref, w4_ref, gs_ref, gd_ref, o_ref):
    ea = ea_ref[...]
    wt = jnp.where(gs_ref[:, 0:1] == gd_ref[:, 0:1], gs_ref[:, 1:2], 0.0)
    for i in range(4):
        P = jnp.dot(ea, w4_ref[i], preferred_element_type=jnp.float32)
        if i == 0:
            G = P
        else:
            L = _leaky(P)
            G = L
            for _ in range(i - 1):
                G = G * L
        o_ref[i] = G * wt


def _powers(ea, conv4W, Gs, Gd):
    return pl.pallas_call(
        _powers_body,
        grid=(80,),
        in_specs=[pl.BlockSpec((2000, 128), lambda i: (i, 0)),
                  pl.BlockSpec((4, 128, 128), lambda i: (0, 0, 0)),
                  pl.BlockSpec((2000, 128), lambda i: (i, 0)),
                  pl.BlockSpec((2000, 128), lambda i: (i, 0))],
        out_specs=pl.BlockSpec((4, 2000, 128), lambda i: (0, i, 0)),
        out_shape=jax.ShapeDtypeStruct((4, E, 128), jnp.float32),
        interpret=_INTERPRET,
    )(ea, conv4W, Gs, Gd)


def _gather_edges_sc(xs1, src, dst):
    mesh = plsc.VectorSubcoreMesh(core_axis_name="c", subcore_axis_name="s")

    @functools.partial(
        pl.kernel, mesh=mesh,
        out_type=[jax.ShapeDtypeStruct((E, 128), jnp.float32),
                  jax.ShapeDtypeStruct((E, 128), jnp.float32)],
        scratch_types=[
            pltpu.VMEM((128,), jnp.int32),
            pltpu.VMEM((128, 128), jnp.float32),
            pltpu.SemaphoreType.DMA,
        ],
    )
    def k(xs1_hbm, src_hbm, dst_hbm, xi_hbm, xj_hbm, idx_v, rows_v, sem):
        c = lax.axis_index("c")
        s = lax.axis_index("s")
        wid = s * 2 + c
        nb = 39 + jnp.where(wid < 2, 1, 0)

        def body(j, carry):
            bi = wid + 32 * j
            pltpu.sync_copy(dst_hbm.at[pl.ds(bi * 128, 128)], idx_v)
            pltpu.async_copy(xs1_hbm.at[idx_v], rows_v, sem).wait()
            pltpu.sync_copy(rows_v, xi_hbm.at[pl.ds(bi * 128, 128)])
            pltpu.sync_copy(src_hbm.at[pl.ds(bi * 128, 128)], idx_v)
            pltpu.async_copy(xs1_hbm.at[idx_v], rows_v, sem).wait()
            pltpu.sync_copy(rows_v, xj_hbm.at[pl.ds(bi * 128, 128)])
            return carry

        lax.fori_loop(0, nb, body, 0)

    return k(xs1, src, dst)


def _edge_weight_sc(src, dst, T):
    mesh = plsc.VectorSubcoreMesh(core_axis_name="c", subcore_axis_name="s")

    @functools.partial(
        pl.kernel, mesh=mesh,
        out_type=[jax.ShapeDtypeStruct((E, 128), jnp.float32),
                  jax.ShapeDtypeStruct((E, 128), jnp.float32)],
        scratch_types=[
            pltpu.VMEM((128,), jnp.int32),
            pltpu.VMEM((128, 128), jnp.float32),
            pltpu.SemaphoreType.DMA,
        ],
    )
    def k(src_hbm, dst_hbm, t_hbm, gs_hbm, gd_hbm, idx_v, rows_v, sem):
        c = lax.axis_index("c")
        s = lax.axis_index("s")
        wid = s * 2 + c
        nb = 39 + jnp.where(wid < 2, 1, 0)

        def body(j, carry):
            bi = wid + 32 * j
            pltpu.sync_copy(src_hbm.at[pl.ds(bi * 128, 128)], idx_v)
            pltpu.async_copy(t_hbm.at[idx_v], rows_v, sem).wait()
            pltpu.sync_copy(rows_v, gs_hbm.at[pl.ds(bi * 128, 128)])
            pltpu.sync_copy(dst_hbm.at[pl.ds(bi * 128, 128)], idx_v)
            pltpu.async_copy(t_hbm.at[idx_v], rows_v, sem).wait()
            pltpu.sync_copy(rows_v, gd_hbm.at[pl.ds(bi * 128, 128)])
            return carry

        lax.fori_loop(0, nb, body, 0)

    return k(src, dst, T)


def _gather_mean_sc(e_pad, x, i0, i1, i2, i3):
    mesh = plsc.VectorSubcoreMesh(core_axis_name="c", subcore_axis_name="s")

    @functools.partial(
        pl.kernel, mesh=mesh,
        out_type=[jax.ShapeDtypeStruct((NHPAD, 128), jnp.float32),
                  jax.ShapeDtypeStruct((NHPAD, 128), jnp.float32)],
        scratch_types=[
            pltpu.VMEM((128,), jnp.int32),
            pltpu.VMEM((128, 128), jnp.float32),
            pltpu.VMEM((128, 128), jnp.float32),
            pltpu.SemaphoreType.DMA,
        ],
    )
    def k(e_hbm, x_hbm, i0_hbm, i1_hbm, i2_hbm, i3_hbm, eh_hbm, xh_hbm,
          idx_v, rows_v, acc_v, sem):
        c = lax.axis_index("c")
        s = lax.axis_index("s")
        wid = s * 2 + c
        nb = 1 + jnp.where(wid < 8, 1, 0)
        idx_refs = [i0_hbm, i1_hbm, i2_hbm, i3_hbm]

        def accumulate(first):
            def rbody(r, carry):
                for g in range(8):
                    v = rows_v[r, pl.ds(g * 16, 16)]
                    if first:
                        acc_v[r, pl.ds(g * 16, 16)] = v
                    else:
                        acc_v[r, pl.ds(g * 16, 16)] = acc_v[r, pl.ds(g * 16, 16)] + v
                return carry
            return rbody

        def scale_store(dst_hbm, bi):
            def rbody(r, carry):
                for g in range(8):
                    acc_v[r, pl.ds(g * 16, 16)] = acc_v[r, pl.ds(g * 16, 16)] * 0.25
                return carry
            lax.fori_loop(0, 128, rbody, 0)
            pltpu.sync_copy(acc_v, dst_hbm.at[pl.ds(bi * 128, 128)])

        def body(j, carry):
            bi = wid + 32 * j
            for table, out in ((e_hbm, eh_hbm), (x_hbm, xh_hbm)):
                for jj in range(4):
                    pltpu.sync_copy(idx_refs[jj].at[pl.ds(bi * 128, 128)], idx_v)
                    pltpu.async_copy(table.at[idx_v], rows_v, sem).wait()
                    lax.fori_loop(0, 128, accumulate(jj == 0), 0)
                scale_store(out, bi)
            return carry

        lax.fori_loop(0, nb, body, 0)

    return k(e_pad, x, i0, i1, i2, i3)


def _scatter_add_sc(SG, dst, zeros):
    mesh = plsc.VectorSubcoreMesh(core_axis_name="c", subcore_axis_name="s")

    @functools.partial(
        pl.kernel, mesh=mesh,
        out_type=jax.ShapeDtypeStruct((2, 4, NPAD, 128), jnp.float32),
        scratch_types=[
            pltpu.VMEM((128,), jnp.int32),
            pltpu.VMEM((128, 128), jnp.float32),
            pltpu.VMEM_SHARED((NPAD, 128), jnp.float32),
            pltpu.SemaphoreType.DMA,
        ],
    )
    def k(sg_hbm, dst_hbm, z_hbm, out_hbm, idx_v, rows_v, acc_sh, sem):
        c = lax.axis_index("c")
        s = lax.axis_index("s")
        wid = s * 2 + c
        nb = 39 + jnp.where(wid < 2, 1, 0)
        for i in range(4):
            pltpu.sync_copy(z_hbm, acc_sh.at[pl.ds(s * 640, 640)])
            plsc.subcore_barrier()

            def body(j, carry):
                bi = wid + 32 * j
                pltpu.sync_copy(dst_hbm.at[pl.ds(bi * 128, 128)], idx_v)
                pltpu.sync_copy(sg_hbm.at[i, pl.ds(bi * 128, 128)], rows_v)
                pltpu.sync_copy(rows_v, acc_sh.at[idx_v], add=True)
                return carry

            lax.fori_loop(0, nb, body, 0)
            plsc.subcore_barrier()
            pltpu.sync_copy(acc_sh.at[pl.ds(s * 640, 640)],
                            out_hbm.at[c, i, pl.ds(s * 640, 640)])
            plsc.subcore_barrier()

    return k(SG, dst, zeros)


def _combine_body(s_ref, cl_ref, al_ref, o_ref):
    acc = jnp.zeros(o_ref.shape, jnp.float32)
    cl = cl_ref[...]
    for k in range(4):
        mk = cl == k
        for i in range(4):
            sk = jnp.where(mk, s_ref[0, i] + s_ref[1, i], 0.0)
            acc = acc + jax.lax.dot_general(
                sk, al_ref[k, i], (((1,), (1,)), ((), ())),
                preferred_element_type=jnp.float32)
    o_ref[...] = acc


def _combine(S, cluster_pad, alpha4):
    return pl.pallas_call(
        _combine_body,
        grid=(8,),
        in_specs=[pl.BlockSpec((2, 4, 1280, 128), lambda i: (0, 0, i, 0)),
                  pl.BlockSpec((1280, 1), lambda i: (i, 0)),
                  pl.BlockSpec((4, 4, 128, 128), lambda i: (0, 0, 0, 0))],
        out_specs=pl.BlockSpec((1280, 128), lambda i: (i, 0)),
        out_shape=jax.ShapeDtypeStruct((NPAD, 128), jnp.float32),
        interpret=_INTERPRET,
    )(S, cluster_pad, alpha4)


def _top4_body(qx_ref, qy_ref, px_ref, py_ref, o0, o1, o2, o3):
    qx = qx_ref[...]
    qy = qy_ref[...]
    dx = qx - px_ref[...]
    dy = qy - py_ref[...]
    d = jnp.sqrt(dx * dx + dy * dy)
    iota = jax.lax.broadcasted_iota(jnp.int32, d.shape, 1)
    outs = [o0, o1, o2, o3]
    for j in range(4):
        m = jnp.min(d, axis=1, keepdims=True)
        idx = jnp.min(jnp.where(d == m, iota, 2147483647), axis=1, keepdims=True)
        outs[j][...] = idx
        if j < 3:
            d = jnp.where(iota == idx, jnp.inf, d)


def _top4(phx, phy, pxp, pyp):
    shp = jax.ShapeDtypeStruct((NHPAD, 1), jnp.int32)
    return pl.pallas_call(
        _top4_body,
        grid=(40,),
        in_specs=[pl.BlockSpec((128, 1), lambda i: (i, 0)),
                  pl.BlockSpec((128, 1), lambda i: (i, 0)),
                  pl.BlockSpec((1, NPAD), lambda i: (0, 0)),
                  pl.BlockSpec((1, NPAD), lambda i: (0, 0))],
        out_specs=[pl.BlockSpec((128, 1), lambda i: (i, 0))] * 4,
        out_shape=[shp, shp, shp, shp],
        interpret=_INTERPRET,
    )(phx, phy, pxp, pyp)


def _final_body(eh_ref, xh_ref, wa_ref, wb_ref, b_ref, o_ref):
    o_ref[...] = (
        jnp.dot(eh_ref[...], wa_ref[...], preferred_element_type=jnp.float32)
        + jnp.dot(xh_ref[...], wb_ref[...], preferred_element_type=jnp.float32)
        + b_ref[...])


def _final(ehp, xhp, W3, b3):
    return pl.pallas_call(
        _final_body,
        grid=(4,),
        in_specs=[pl.BlockSpec((1280, 128), lambda i: (i, 0)),
                  pl.BlockSpec((1280, 128), lambda i: (i, 0)),
                  pl.BlockSpec((128, 128), lambda i: (0, 0)),
                  pl.BlockSpec((128, 128), lambda i: (0, 0)),
                  pl.BlockSpec((1, 128), lambda i: (0, 0))],
        out_specs=pl.BlockSpec((1280, 128), lambda i: (i, 0)),
        out_shape=jax.ShapeDtypeStruct((NHPAD, 128), jnp.float32),
        interpret=_INTERPRET,
    )(ehp, xhp, W3[:128], W3[128:], b3.reshape(1, 128))


def kernel(x, edge_index, edge_attr, pos, edge_index_high, edge_attr_high, pos_high, W_sim, b_sim, W_e1, b_e1, W_e2, b_e2, conv1W, alpha1, conv4W, alpha4, W3, b3):
    n = x.shape[0]
    src, dst = edge_index[0], edge_index[1]

    xp = jnp.concatenate([x, pos], axis=1)
    xs1 = _xs1(xp, W_sim, b_sim)

    XI, XJ = _gather_edges_sc(xs1, src, dst)

    h = _edge_mlp(XI, XJ, W_e1, b_e1, W_e2, b_e2)

    xs2 = jnp.zeros((n, 64), jnp.float32).at[dst].max(h)

    init = jax.random.randint(jax.random.key(42), (4,), 0, n)
    c0 = xs2[init]
    cluster = _kmeans(xs2, c0)[:, 0]

    deg = jax.ops.segment_sum(jnp.ones((src.shape[0],), jnp.float32), src, num_segments=n)
    rdeg_pad = jnp.pad(1.0 / jnp.maximum(deg, 1.0), (0, NPAD - n))
    cl_pad1d = jnp.pad(cluster, (0, NPAD - n))
    T = jnp.zeros((NPAD, 128), jnp.float32)
    T = T.at[:, 0].set(cl_pad1d.astype(jnp.float32)).at[:, 1].set(rdeg_pad)
    Gs, Gd = _edge_weight_sc(src, dst, T)

    SG = _powers(edge_attr, conv4W, Gs, Gd)

    S = _scatter_add_sc(SG, dst, jnp.zeros((640, 128), jnp.float32))

    cluster_pad = jnp.pad(cluster, (0, NPAD - n), constant_values=-1).reshape(NPAD, 1)
    e = _combine(S, cluster_pad, alpha4)

    phx = jnp.pad(pos_high[:, 0], (0, NHPAD - NH)).reshape(NHPAD, 1)
    phy = jnp.pad(pos_high[:, 1], (0, NHPAD - NH)).reshape(NHPAD, 1)
    pxp = jnp.pad(pos[:, 0], (0, NPAD - n), constant_values=1e30).reshape(1, NPAD)
    pyp = jnp.pad(pos[:, 1], (0, NPAD - n), constant_values=1e30).reshape(1, NPAD)
    i0, i1, i2, i3 = _top4(phx, phy, pxp, pyp)

    ehp, xhp = _gather_mean_sc(e, x, i0.reshape(NHPAD), i1.reshape(NHPAD),
                               i2.reshape(NHPAD), i3.reshape(NHPAD))

    out = _final(ehp, xhp, W3, b3)
    return out[:NH]


# speedup vs baseline: 9.7650x; 1.0000x over previous
import functools

import jax
import jax.numpy as jnp
from jax import lax
from jax.experimental import pallas as pl
from jax.experimental.pallas import tpu as pltpu
from jax.experimental.pallas import tpu_sc as plsc

N = 10000
E = 160000
D = 128
NPAD = 10240
NH = 5000
NHPAD = 5120


def _leaky(v):
    return jnp.where(v >= 0, v, 0.1 * v)


def _xs1_body(xp_ref, w_ref, b_ref, o_ref):
    o_ref[...] = jax.nn.relu(
        jnp.dot(xp_ref[...], w_ref[...], preferred_element_type=jnp.float32)
        + b_ref[...])


def _xs1(xp, W_sim, b_sim):
    return pl.pallas_call(
        _xs1_body,
        grid=(5,),
        in_specs=[pl.BlockSpec((2000, 130), lambda i: (i, 0)),
                  pl.BlockSpec((130, 128), lambda i: (0, 0)),
                  pl.BlockSpec((1, 128), lambda i: (0, 0))],
        out_specs=pl.BlockSpec((2000, 128), lambda i: (i, 0)),
        out_shape=jax.ShapeDtypeStruct((N, 128), jnp.float32),
    )(xp, W_sim, b_sim.reshape(1, 128))


def _edge_mlp_body(xi_ref, xj_ref, w1_ref, b1_ref, w2_ref, b2_ref, o_ref):
    xi = xi_ref[...]
    ec = jnp.concatenate([xi, xj_ref[...] - xi], axis=1)
    t = jax.nn.relu(
        jnp.dot(ec, w1_ref[...], preferred_element_type=jnp.float32)
        + b1_ref[...])
    o_ref[...] = (jnp.dot(t, w2_ref[...], preferred_element_type=jnp.float32)
                  + b2_ref[...])


def _edge_mlp(XI, XJ, W_e1, b_e1, W_e2, b_e2):
    return pl.pallas_call(
        _edge_mlp_body,
        grid=(40,),
        in_specs=[pl.BlockSpec((4000, 128), lambda i: (i, 0)),
                  pl.BlockSpec((4000, 128), lambda i: (i, 0)),
                  pl.BlockSpec((256, 64), lambda i: (0, 0)),
                  pl.BlockSpec((1, 64), lambda i: (0, 0)),
                  pl.BlockSpec((64, 64), lambda i: (0, 0)),
                  pl.BlockSpec((1, 64), lambda i: (0, 0))],
        out_specs=pl.BlockSpec((4000, 64), lambda i: (i, 0)),
        out_shape=jax.ShapeDtypeStruct((E, 64), jnp.float32),
    )(XI, XJ, W_e1, b_e1.reshape(1, 64), W_e2, b_e2.reshape(1, 64))


def _kmeans_body(x_ref, c0_ref, cl_ref):
    Xv = x_ref[...]

    def cond(s):
        i, c, a, conv = s
        return (i < 300) & jnp.logical_not(conv)

    def body(s):
        i, c, a, conv = s
        ds = []
        for k in range(4):
            diff = Xv - c[k, :][None, :]
            ds.append(jnp.sum(diff * diff, axis=1, keepdims=True))
        a = jnp.zeros((N, 1), jnp.int32)
        dm = ds[0]
        for k in range(1, 4):
            m = ds[k] < dm
            a = jnp.where(m, k, a)
            dm = jnp.where(m, ds[k], dm)
        rows = []
        cnts = []
        for k in range(4):
            mk = a == k
            rows.append(jnp.sum(jnp.where(mk, Xv, 0.0), axis=0, keepdims=True))
            cnts.append(jnp.sum(jnp.where(mk, 1.0, 0.0), axis=0, keepdims=True))
        sums = jnp.concatenate(rows, axis=0)
        cnt = jnp.concatenate(cnts, axis=0)[:, :1]
        newc = sums / jnp.maximum(cnt, 1.0)
        dc = newc - c
        conv = jnp.sqrt(jnp.sum(dc * dc)) < 1e-4
        return (i + 1, newc, a, conv)

    _, _, a, _ = jax.lax.while_loop(
        cond, body,
        (jnp.int32(0), c0_ref[...], jnp.zeros((N, 1), jnp.int32), jnp.bool_(False)))
    cl_ref[...] = a


def _kmeans(xs2, c0):
    return pl.pallas_call(
        _kmeans_body,
        out_shape=jax.ShapeDtypeStruct((N, 1), jnp.int32),
    )(xs2, c0)


def _powers_body(ea_ref, w4_ref, gs_ref, gd_ref, o_ref):
    ea = ea_ref[...]
    wt = jnp.where(gs_ref[:, 0:1] == gd_ref[:, 0:1], gs_ref[:, 1:2], 0.0)
    for i in range(4):
        P = jnp.dot(ea, w4_ref[i], preferred_element_type=jnp.float32)
        if i == 0:
            G = P
        else:
            L = _leaky(P)
            G = L
            for _ in range(i - 1):
                G = G * L
        o_ref[i] = G * wt


def _powers(ea, conv4W, Gs, Gd):
    return pl.pallas_call(
        _powers_body,
        grid=(80,),
        in_specs=[pl.BlockSpec((2000, 128), lambda i: (i, 0)),
                  pl.BlockSpec((4, 128, 128), lambda i: (0, 0, 0)),
                  pl.BlockSpec((2000, 128), lambda i: (i, 0)),
                  pl.BlockSpec((2000, 128), lambda i: (i, 0))],
        out_specs=pl.BlockSpec((4, 2000, 128), lambda i: (0, i, 0)),
        out_shape=jax.ShapeDtypeStruct((4, E, 128), jnp.float32),
    )(ea, conv4W, Gs, Gd)


def _gather_edges_sc(xs1, src, dst):
    mesh = plsc.VectorSubcoreMesh(core_axis_name="c", subcore_axis_name="s")

    @functools.partial(
        pl.kernel, mesh=mesh,
        out_type=[jax.ShapeDtypeStruct((E, 128), jnp.float32),
                  jax.ShapeDtypeStruct((E, 128), jnp.float32)],
        scratch_types=[
            pltpu.VMEM((128,), jnp.int32),
            pltpu.VMEM((128, 128), jnp.float32),
            pltpu.SemaphoreType.DMA,
        ],
    )
    def k(xs1_hbm, src_hbm, dst_hbm, xi_hbm, xj_hbm, idx_v, rows_v, sem):
        c = lax.axis_index("c")
        s = lax.axis_index("s")
        wid = s * 2 + c
        nb = 39 + jnp.where(wid < 2, 1, 0)

        def body(j, carry):
            bi = wid + 32 * j
            pltpu.sync_copy(dst_hbm.at[pl.ds(bi * 128, 128)], idx_v)
            pltpu.async_copy(xs1_hbm.at[idx_v], rows_v, sem).wait()
            pltpu.sync_copy(rows_v, xi_hbm.at[pl.ds(bi * 128, 128)])
            pltpu.sync_copy(src_hbm.at[pl.ds(bi * 128, 128)], idx_v)
            pltpu.async_copy(xs1_hbm.at[idx_v], rows_v, sem).wait()
            pltpu.sync_copy(rows_v, xj_hbm.at[pl.ds(bi * 128, 128)])
            return carry

        lax.fori_loop(0, nb, body, 0)

    return k(xs1, src, dst)


def _edge_weight_sc(src, dst, T):
    mesh = plsc.VectorSubcoreMesh(core_axis_name="c", subcore_axis_name="s")

    @functools.partial(
        pl.kernel, mesh=mesh,
        out_type=[jax.ShapeDtypeStruct((E, 128), jnp.float32),
                  jax.ShapeDtypeStruct((E, 128), jnp.float32)],
        scratch_types=[
            pltpu.VMEM((128,), jnp.int32),
            pltpu.VMEM((128, 128), jnp.float32),
            pltpu.SemaphoreType.DMA,
        ],
    )
    def k(src_hbm, dst_hbm, t_hbm, gs_hbm, gd_hbm, idx_v, rows_v, sem):
        c = lax.axis_index("c")
        s = lax.axis_index("s")
        wid = s * 2 + c
        nb = 39 + jnp.where(wid < 2, 1, 0)

        def body(j, carry):
            bi = wid + 32 * j
            pltpu.sync_copy(src_hbm.at[pl.ds(bi * 128, 128)], idx_v)
            pltpu.async_copy(t_hbm.at[idx_v], rows_v, sem).wait()
            pltpu.sync_copy(rows_v, gs_hbm.at[pl.ds(bi * 128, 128)])
            pltpu.sync_copy(dst_hbm.at[pl.ds(bi * 128, 128)], idx_v)
            pltpu.async_copy(t_hbm.at[idx_v], rows_v, sem).wait()
            pltpu.sync_copy(rows_v, gd_hbm.at[pl.ds(bi * 128, 128)])
            return carry

        lax.fori_loop(0, nb, body, 0)

    return k(src, dst, T)


def _gather_mean_sc(e_pad, x, i0, i1, i2, i3):
    mesh = plsc.VectorSubcoreMesh(core_axis_name="c", subcore_axis_name="s")

    @functools.partial(
        pl.kernel, mesh=mesh,
        out_type=[jax.ShapeDtypeStruct((NHPAD, 128), jnp.float32),
                  jax.ShapeDtypeStruct((NHPAD, 128), jnp.float32)],
        scratch_types=[
            pltpu.VMEM((128,), jnp.int32),
            pltpu.VMEM((128, 128), jnp.float32),
            pltpu.VMEM((128, 128), jnp.float32),
            pltpu.SemaphoreType.DMA,
        ],
    )
    def k(e_hbm, x_hbm, i0_hbm, i1_hbm, i2_hbm, i3_hbm, eh_hbm, xh_hbm,
          idx_v, rows_v, acc_v, sem):
        c = lax.axis_index("c")
        s = lax.axis_index("s")
        wid = s * 2 + c
        nb = 1 + jnp.where(wid < 8, 1, 0)
        idx_refs = [i0_hbm, i1_hbm, i2_hbm, i3_hbm]

        def accumulate(first):
            def rbody(r, carry):
                for g in range(8):
                    v = rows_v[r, pl.ds(g * 16, 16)]
                    if first:
                        acc_v[r, pl.ds(g * 16, 16)] = v
                    else:
                        acc_v[r, pl.ds(g * 16, 16)] = acc_v[r, pl.ds(g * 16, 16)] + v
                return carry
            return rbody

        def scale_store(dst_hbm, bi):
            def rbody(r, carry):
                for g in range(8):
                    acc_v[r, pl.ds(g * 16, 16)] = acc_v[r, pl.ds(g * 16, 16)] * 0.25
                return carry
            lax.fori_loop(0, 128, rbody, 0)
            pltpu.sync_copy(acc_v, dst_hbm.at[pl.ds(bi * 128, 128)])

        def body(j, carry):
            bi = wid + 32 * j
            for table, out in ((e_hbm, eh_hbm), (x_hbm, xh_hbm)):
                for jj in range(4):
                    pltpu.sync_copy(idx_refs[jj].at[pl.ds(bi * 128, 128)], idx_v)
                    pltpu.async_copy(table.at[idx_v], rows_v, sem).wait()
                    lax.fori_loop(0, 128, accumulate(jj == 0), 0)
                scale_store(out, bi)
            return carry

        lax.fori_loop(0, nb, body, 0)

    return k(e_pad, x, i0, i1, i2, i3)


def _scatter_add_sc(SG, dst, zeros):
    mesh = plsc.VectorSubcoreMesh(core_axis_name="c", subcore_axis_name="s")

    @functools.partial(
        pl.kernel, mesh=mesh,
        out_type=jax.ShapeDtypeStruct((2, 4, NPAD, 128), jnp.float32),
        scratch_types=[
            pltpu.VMEM((128,), jnp.int32),
            pltpu.VMEM((128, 128), jnp.float32),
            pltpu.VMEM_SHARED((NPAD, 128), jnp.float32),
            pltpu.SemaphoreType.DMA,
        ],
    )
    def k(sg_hbm, dst_hbm, z_hbm, out_hbm, idx_v, rows_v, acc_sh, sem):
        c = lax.axis_index("c")
        s = lax.axis_index("s")
        wid = s * 2 + c
        nb = 39 + jnp.where(wid < 2, 1, 0)
        for i in range(4):
            pltpu.sync_copy(z_hbm, acc_sh.at[pl.ds(s * 640, 640)])
            plsc.subcore_barrier()

            def body(j, carry):
                bi = wid + 32 * j
                pltpu.sync_copy(dst_hbm.at[pl.ds(bi * 128, 128)], idx_v)
                pltpu.sync_copy(sg_hbm.at[i, pl.ds(bi * 128, 128)], rows_v)
                pltpu.sync_copy(rows_v, acc_sh.at[idx_v], add=True)
                return carry

            lax.fori_loop(0, nb, body, 0)
            plsc.subcore_barrier()
            pltpu.sync_copy(acc_sh.at[pl.ds(s * 640, 640)],
                            out_hbm.at[c, i, pl.ds(s * 640, 640)])
            plsc.subcore_barrier()

    return k(SG, dst, zeros)


def _combine_body(s_ref, cl_ref, al_ref, o_ref):
    acc = jnp.zeros(o_ref.shape, jnp.float32)
    cl = cl_ref[...]
    for k in range(4):
        mk = cl == k
        for i in range(4):
            sk = jnp.where(mk, s_ref[0, i] + s_ref[1, i], 0.0)
            acc = acc + jax.lax.dot_general(
                sk, al_ref[k, i], (((1,), (1,)), ((), ())),
                preferred_element_type=jnp.float32)
    o_ref[...] = acc


def _combine(S, cluster_pad, alpha4):
    return pl.pallas_call(
        _combine_body,
        grid=(8,),
        in_specs=[pl.BlockSpec((2, 4, 1280, 128), lambda i: (0, 0, i, 0)),
                  pl.BlockSpec((1280, 1), lambda i: (i, 0)),
                  pl.BlockSpec((4, 4, 128, 128), lambda i: (0, 0, 0, 0))],
        out_specs=pl.BlockSpec((1280, 128), lambda i: (i, 0)),
        out_shape=jax.ShapeDtypeStruct((NPAD, 128), jnp.float32),
    )(S, cluster_pad, alpha4)


def _top4_body(qx_ref, qy_ref, px_ref, py_ref, o0, o1, o2, o3):
    qx = qx_ref[...]
    qy = qy_ref[...]
    dx = qx - px_ref[...]
    dy = qy - py_ref[...]
    d = jnp.sqrt(dx * dx + dy * dy)
    iota = jax.lax.broadcasted_iota(jnp.int32, d.shape, 1)
    outs = [o0, o1, o2, o3]
    for j in range(4):
        m = jnp.min(d, axis=1, keepdims=True)
        idx = jnp.min(jnp.where(d == m, iota, 2147483647), axis=1, keepdims=True)
        outs[j][...] = idx
        if j < 3:
            d = jnp.where(iota == idx, jnp.inf, d)


def _top4(phx, phy, pxp, pyp):
    shp = jax.ShapeDtypeStruct((NHPAD, 1), jnp.int32)
    return pl.pallas_call(
        _top4_body,
        grid=(40,),
        in_specs=[pl.BlockSpec((128, 1), lambda i: (i, 0)),
                  pl.BlockSpec((128, 1), lambda i: (i, 0)),
                  pl.BlockSpec((1, NPAD), lambda i: (0, 0)),
                  pl.BlockSpec((1, NPAD), lambda i: (0, 0))],
        out_specs=[pl.BlockSpec((128, 1), lambda i: (i, 0))] * 4,
        out_shape=[shp, shp, shp, shp],
    )(phx, phy, pxp, pyp)


def _final_body(eh_ref, xh_ref, wa_ref, wb_ref, b_ref, o_ref):
    o_ref[...] = (
        jnp.dot(eh_ref[...], wa_ref[...], preferred_element_type=jnp.float32)
        + jnp.dot(xh_ref[...], wb_ref[...], preferred_element_type=jnp.float32)
        + b_ref[...])


def _final(ehp, xhp, W3, b3):
    return pl.pallas_call(
        _final_body,
        grid=(4,),
        in_specs=[pl.BlockSpec((1280, 128), lambda i: (i, 0)),
                  pl.BlockSpec((1280, 128), lambda i: (i, 0)),
                  pl.BlockSpec((128, 128), lambda i: (0, 0)),
                  pl.BlockSpec((128, 128), lambda i: (0, 0)),
                  pl.BlockSpec((1, 128), lambda i: (0, 0))],
        out_specs=pl.BlockSpec((1280, 128), lambda i: (i, 0)),
        out_shape=jax.ShapeDtypeStruct((NHPAD, 128), jnp.float32),
    )(ehp, xhp, W3[:128], W3[128:], b3.reshape(1, 128))


def kernel(x, edge_index, edge_attr, pos, edge_index_high, edge_attr_high, pos_high, W_sim, b_sim, W_e1, b_e1, W_e2, b_e2, conv1W, alpha1, conv4W, alpha4, W3, b3):
    n = x.shape[0]
    src, dst = edge_index[0], edge_index[1]

    xp = jnp.concatenate([x, pos], axis=1)
    xs1 = _xs1(xp, W_sim, b_sim)

    XI, XJ = _gather_edges_sc(xs1, src, dst)

    h = _edge_mlp(XI, XJ, W_e1, b_e1, W_e2, b_e2)

    xs2 = jnp.zeros((n, 64), jnp.float32).at[dst].max(h)

    init = jax.random.randint(jax.random.key(42), (4,), 0, n)
    c0 = xs2[init]
    cluster = _kmeans(xs2, c0)[:, 0]

    deg = jax.ops.segment_sum(jnp.ones((src.shape[0],), jnp.float32), src, num_segments=n)
    rdeg_pad = jnp.pad(1.0 / jnp.maximum(deg, 1.0), (0, NPAD - n))
    cl_pad1d = jnp.pad(cluster, (0, NPAD - n))
    T = jnp.zeros((NPAD, 128), jnp.float32)
    T = T.at[:, 0].set(cl_pad1d.astype(jnp.float32)).at[:, 1].set(rdeg_pad)
    Gs, Gd = _edge_weight_sc(src, dst, T)

    SG = _powers(edge_attr, conv4W, Gs, Gd)

    S = _scatter_add_sc(SG, dst, jnp.zeros((640, 128), jnp.float32))

    cluster_pad = jnp.pad(cluster, (0, NPAD - n), constant_values=-1).reshape(NPAD, 1)
    e = _combine(S, cluster_pad, alpha4)

    phx = jnp.pad(pos_high[:, 0], (0, NHPAD - NH)).reshape(NHPAD, 1)
    phy = jnp.pad(pos_high[:, 1], (0, NHPAD - NH)).reshape(NHPAD, 1)
    pxp = jnp.pad(pos[:, 0], (0, NPAD - n), constant_values=1e30).reshape(1, NPAD)
    pyp = jnp.pad(pos[:, 1], (0, NPAD - n), constant_values=1e30).reshape(1, NPAD)
    i0, i1, i2, i3 = _top4(phx, phy, pxp, pyp)

    ehp, xhp = _gather_mean_sc(e, x, i0.reshape(NHPAD), i1.reshape(NHPAD),
                               i2.reshape(NHPAD), i3.reshape(NHPAD))

    out = _final(ehp, xhp, W3, b3)
    return out[:NH]
